# baseline (speedup 1.0000x reference)
import jax
import jax.numpy as jnp
from jax import lax
from jax.experimental import pallas as pl
from jax.experimental.pallas import tpu as pltpu
from jax.experimental.pallas import tpu_sc as plsc

N = 10000
E = 10000
P = 160000
IN_DIM = 256
HID = 128
OUT = 32
ATT_HID = 16
EPS = 1e-5

NS = 16
K = 80
NK = P // NS // K
assert NK % 2 == 1
NP_PAD = 10240
RN = NP_PAD // NS
RC = 128
NRC = RN // RC
BLK = 1000


def _make_smooth(D, do_relu, nh):
    Dh = D // nh
    CC = Dh // 16
    mesh = plsc.VectorSubcoreMesh(core_axis_name="c", subcore_axis_name="s")
    fdt = jnp.float32
    out_type = tuple(
        jax.ShapeDtypeStruct((2 * NP_PAD, Dh), fdt) for _ in range(nh)
    )
    scratch = [
        pltpu.VMEM((NK, K), jnp.int32),
        pltpu.VMEM((NK, K), jnp.int32),
        pltpu.VMEM((NK, K), jnp.int32),
        pltpu.VMEM((K, Dh), fdt),
        pltpu.VMEM((K, Dh), fdt),
        pltpu.VMEM((RC, Dh), fdt),
        pltpu.VMEM((RC, Dh), fdt),
        pltpu.VMEM((RN,), fdt),
        pltpu.VMEM((2, D), fdt),
        pltpu.VMEM((K,), fdt),
        pltpu.VMEM_SHARED((NP_PAD, Dh), fdt),
        pltpu.VMEM_SHARED((NP_PAD, Dh), fdt),
        pltpu.VMEM_SHARED((NP_PAD,), fdt),
        pltpu.VMEM_SHARED((NP_PAD,), fdt),
        pltpu.SemaphoreType.DMA,
        pltpu.SemaphoreType.DMA,
    ]

    def body(*refs):
        hparts = refs[:nh]
        vil_h, vig_h, eil_h, aff_h = refs[nh:nh + 4]
        vos = refs[nh + 4:nh + 4 + nh]
        (vil, vig, eil, rowsA, rowsB, nb, zb, cntb, stb, onesb,
         eacc, vacc, ecnt, vcnt, semA, semB) = refs[nh + 4 + nh:]

        def gsc_pipe(src, gi, scatter):
            pltpu.async_copy(src.at[gi.at[0]], rowsA, semA)

            def step(j, _):
                k = 2 * j
                pltpu.make_async_copy(src.at[gi.at[k]], rowsA, semA).wait()
                pltpu.async_copy(src.at[gi.at[k + 1]], rowsB, semB)
                scatter(rowsA, k)
                pltpu.make_async_copy(
                    src.at[gi.at[k + 1]], rowsB, semB).wait()
                pltpu.async_copy(src.at[gi.at[k + 2]], rowsA, semA)
                scatter(rowsB, k + 1)
                return 0

            lax.fori_loop(0, (NK - 1) // 2, step, 0)
            kl = NK - 1
            pltpu.make_async_copy(src.at[gi.at[kl]], rowsA, semA).wait()
            scatter(rowsA, kl)

        c = lax.axis_index("c")
        s = lax.axis_index("s")
        z16 = jnp.zeros((16,), fdt)
        o16 = jnp.ones((16,), fdt)
        r0 = s * RN
        ob0 = c * NP_PAD
        cb0 = c * (NS * NK) + s * NK

        def zb_fill(i, _):
            zb[i // CC, pl.ds((i % CC) * 16, 16)] = z16
            return 0
        lax.fori_loop(0, RC * CC, zb_fill, 0)

        def ones_fill(i, _):
            onesb[pl.ds(i * 16, 16)] = o16
            return 0
        lax.fori_loop(0, K // 16, ones_fill, 0)

        def cz_fill(i, _):
            cntb[pl.ds(i * 16, 16)] = z16
            return 0
        lax.fori_loop(0, RN // 16, cz_fill, 0)

        for j in range(NRC):
            pltpu.sync_copy(zb, eacc.at[pl.ds(r0 + j * RC, RC)])
            pltpu.sync_copy(zb, vacc.at[pl.ds(r0 + j * RC, RC)])
        pltpu.sync_copy(cntb, ecnt.at[pl.ds(r0, RN)])
        pltpu.sync_copy(cntb, vcnt.at[pl.ds(r0, RN)])

        pltpu.sync_copy(vil_h.at[pl.ds(cb0, NK)], vil)
        pltpu.sync_copy(vig_h.at[pl.ds(cb0, NK)], vig)
        pltpu.sync_copy(eil_h.at[pl.ds(cb0, NK)], eil)
        pltpu.sync_copy(aff_h.at[pl.ds(2 * c, 2)], stb)

        plsc.subcore_barrier()

        for h in range(nh):
            hcat = hparts[h]
            vo = vos[h]
            col0 = h * Dh

            if h == 0:
                def p1s(buf, k):
                    pltpu.sync_copy(buf, eacc.at[eil.at[k]], add=True)
                    pltpu.sync_copy(onesb, ecnt.at[eil.at[k]], add=True)
                    pltpu.sync_copy(onesb, vcnt.at[vil.at[k]], add=True)
            else:
                def p1s(buf, k):
                    pltpu.sync_copy(buf, eacc.at[eil.at[k]], add=True)
            gsc_pipe(hcat, vig, p1s)
            plsc.subcore_barrier()

            pltpu.sync_copy(ecnt.at[pl.ds(r0, RN)], cntb)
            for j in range(NRC):
                rb = r0 + j * RC
                pltpu.sync_copy(eacc.at[pl.ds(rb, RC)], nb)

                def enorm_grp(g, _):
                    cvg = cntb[pl.ds(j * RC + g * 16, 16)]
                    invg = 1.0 / jnp.maximum(cvg, 1.0)
                    tmg = jnp.where(cvg > 0.0, 1.0, 0.0)
                    iota = lax.iota(jnp.int32, 16)

                    def enorm_row(r2, _):
                        sel = iota == r2
                        inv = jnp.sum(jnp.where(sel, invg, 0.0))
                        tm = jnp.sum(jnp.where(sel, tmg, 0.0))
                        r = g * 16 + r2
                        for cc in range(CC):
                            v = nb[r, pl.ds(cc * 16, 16)]
                            sv = stb[0, pl.ds(col0 + cc * 16, 16)]
                            tv = stb[1, pl.ds(col0 + cc * 16, 16)]
                            nb[r, pl.ds(cc * 16, 16)] = v * sv * inv + tv * tm
                        return 0
                    lax.fori_loop(0, 16, enorm_row, 0)
                    return 0
                lax.fori_loop(0, RC // 16, enorm_grp, 0)

                pltpu.sync_copy(nb, eacc.at[pl.ds(rb, RC)])
            plsc.subcore_barrier()

            def p3s(buf, k):
                pltpu.sync_copy(buf, vacc.at[vil.at[k]], add=True)
            gsc_pipe(eacc, eil, p3s)
            plsc.subcore_barrier()

            pltpu.sync_copy(vcnt.at[pl.ds(r0, RN)], cntb)
            for j in range(NRC):
                rb = r0 + j * RC
                pltpu.sync_copy(vacc.at[pl.ds(rb, RC)], nb)
                if h < nh - 1:
                    pltpu.sync_copy(zb, vacc.at[pl.ds(rb, RC)])
                    pltpu.sync_copy(zb, eacc.at[pl.ds(rb, RC)])

                def vnorm_grp(g, _):
                    cvg = cntb[pl.ds(j * RC + g * 16, 16)]
                    invg = 1.0 / jnp.maximum(cvg, 1.0)
                    iota = lax.iota(jnp.int32, 16)

                    def vnorm_row(r2, _):
                        inv = jnp.sum(jnp.where(iota == r2, invg, 0.0))
                        r = g * 16 + r2
                        for cc in range(CC):
                            v = nb[r, pl.ds(cc * 16, 16)] * inv
                            if do_relu:
                                v = jnp.maximum(v, 0.0)
                            nb[r, pl.ds(cc * 16, 16)] = v
                        return 0
                    lax.fori_loop(0, 16, vnorm_row, 0)
                    return 0
                lax.fori_loop(0, RC // 16, vnorm_grp, 0)

                pltpu.sync_copy(nb, vo.at[pl.ds(ob0 + rb, RC)])
            if h < nh - 1:
                plsc.subcore_barrier()

    return pl.kernel(
        body, out_type=out_type, mesh=mesh, scratch_types=scratch,
        compiler_params=pltpu.CompilerParams(
            needs_layout_passes=False, use_tc_tiling_on_sc=False))


_smooth_hid = _make_smooth(HID, True, 4)
_smooth_out = _make_smooth(OUT, False, 1)


def _enc1_body(x_ref, w_ref, hs_ref, hf_ref, st_ref):
    i = pl.program_id(0)
    h = jnp.dot(x_ref[...], w_ref[...], preferred_element_type=jnp.float32)
    hs_ref[...] = h[:, :HID]
    hf_ref[...] = h[:, HID:]
    su = jnp.sum(h, axis=0, keepdims=True)
    sq = jnp.sum(h * h, axis=0, keepdims=True)
    st = jnp.concatenate([su, sq, jnp.zeros((6, 2 * HID), jnp.float32)], axis=0)

    @pl.when(i == 0)
    def _():
        st_ref[...] = jnp.zeros_like(st_ref)

    st_ref[...] += st


def _enc1(x, w):
    return pl.pallas_call(
        _enc1_body,
        grid=(N // BLK,),
        in_specs=[
            pl.BlockSpec((BLK, IN_DIM), lambda i: (i, 0)),
            pl.BlockSpec((IN_DIM, 2 * HID), lambda i: (0, 0)),
        ],
        out_specs=[
            pl.BlockSpec((BLK, HID), lambda i: (i, 0)),
            pl.BlockSpec((BLK, HID), lambda i: (i, 0)),
            pl.BlockSpec((8, 2 * HID), lambda i: (0, 0)),
        ],
        out_shape=[
            jax.ShapeDtypeStruct((N, HID), jnp.float32),
            jax.ShapeDtypeStruct((N, HID), jnp.float32),
            jax.ShapeDtypeStruct((8, 2 * HID), jnp.float32),
        ],
    )(x, w)


def _enc2_body(vs_ref, vf_ref, ws_ref, wf_ref, hs_ref, hf_ref, sts_ref, stf_ref):
    i = pl.program_id(0)
    h2s = jnp.dot(vs_ref[...], ws_ref[...], preferred_element_type=jnp.float32)
    h2f = jnp.dot(vf_ref[...], wf_ref[...], preferred_element_type=jnp.float32)
    hs_ref[...] = h2s
    hf_ref[...] = h2f
    for h2, st_ref in ((h2s, sts_ref), (h2f, stf_ref)):
        su = jnp.sum(h2, axis=0, keepdims=True)
        sq = jnp.sum(h2 * h2, axis=0, keepdims=True)
        st = jnp.concatenate([su, sq, jnp.zeros((6, OUT), jnp.float32)], axis=0)

        @pl.when(i == 0)
        def _():
            st_ref[...] = jnp.zeros_like(st_ref)

        st_ref[...] += st


def _enc2(vs_p, vf_p, w2s, w2f):
    return pl.pallas_call(
        _enc2_body,
        grid=(N // BLK,),
        in_specs=[
            pl.BlockSpec((BLK, HID), lambda i: (i, 0)),
            pl.BlockSpec((BLK, HID), lambda i: (i, 0)),
            pl.BlockSpec((HID, OUT), lambda i: (0, 0)),
            pl.BlockSpec((HID, OUT), lambda i: (0, 0)),
        ],
        out_specs=[
            pl.BlockSpec((BLK, OUT), lambda i: (i, 0)),
            pl.BlockSpec((BLK, OUT), lambda i: (i, 0)),
            pl.BlockSpec((8, OUT), lambda i: (0, 0)),
            pl.BlockSpec((8, OUT), lambda i: (0, 0)),
        ],
        out_shape=[
            jax.ShapeDtypeStruct((N, OUT), jnp.float32),
            jax.ShapeDtypeStruct((N, OUT), jnp.float32),
            jax.ShapeDtypeStruct((8, OUT), jnp.float32),
            jax.ShapeDtypeStruct((8, OUT), jnp.float32),
        ],
    )(vs_p, vf_p, w2s, w2f)


def _fuse_body(zs_ref, zf_ref, aw1_ref, ab1_ref, aw2_ref,
               dw1_ref, db1_ref, dw2_ref, db2_ref,
               z_ref, zso_ref, zfo_ref, xh_ref):
    zs = zs_ref[...]
    zf = zf_ref[...]
    ab1 = ab1_ref[0:1, :]
    ts = jnp.tanh(jnp.dot(zs, aw1_ref[...], preferred_element_type=jnp.float32) + ab1)
    tf = jnp.tanh(jnp.dot(zf, aw1_ref[...], preferred_element_type=jnp.float32) + ab1)
    aw2 = aw2_ref[0:1, :]
    ws = jnp.sum(ts * aw2, axis=1, keepdims=True)
    wf = jnp.sum(tf * aw2, axis=1, keepdims=True)
    m = jnp.maximum(ws, wf)
    es = jnp.exp(ws - m)
    ef = jnp.exp(wf - m)
    den = es + ef
    z = (es / den) * zs + (ef / den) * zf
    z_ref[...] = z
    zso_ref[...] = zs
    zfo_ref[...] = zf
    hd = jnp.maximum(
        jnp.dot(z, dw1_ref[...], preferred_element_type=jnp.float32) + db1_ref[0:1, :], 0.0)
    xh_ref[...] = jnp.dot(hd, dw2_ref[...], preferred_element_type=jnp.float32) + db2_ref[0:1, :]


def _fuse(zs_p, zf_p, att, dec):
    def pad8(v):
        return jnp.broadcast_to(v.reshape(1, -1), (8, v.size))

    return pl.pallas_call(
        _fuse_body,
        grid=(N // BLK,),
        in_specs=[
            pl.BlockSpec((BLK, OUT), lambda i: (i, 0)),
            pl.BlockSpec((BLK, OUT), lambda i: (i, 0)),
            pl.BlockSpec((OUT, ATT_HID), lambda i: (0, 0)),
            pl.BlockSpec((8, ATT_HID), lambda i: (0, 0)),
            pl.BlockSpec((8, ATT_HID), lambda i: (0, 0)),
            pl.BlockSpec((OUT, HID), lambda i: (0, 0)),
            pl.BlockSpec((8, HID), lambda i: (0, 0)),
            pl.BlockSpec((HID, IN_DIM), lambda i: (0, 0)),
            pl.BlockSpec((8, IN_DIM), lambda i: (0, 0)),
        ],
        out_specs=[
            pl.BlockSpec((BLK, OUT), lambda i: (i, 0)),
            pl.BlockSpec((BLK, OUT), lambda i: (i, 0)),
            pl.BlockSpec((BLK, OUT), lambda i: (i, 0)),
            pl.BlockSpec((BLK, IN_DIM), lambda i: (i, 0)),
        ],
        out_shape=[
            jax.ShapeDtypeStruct((N, OUT), jnp.float32),
            jax.ShapeDtypeStruct((N, OUT), jnp.float32),
            jax.ShapeDtypeStruct((N, OUT), jnp.float32),
            jax.ShapeDtypeStruct((N, IN_DIM), jnp.float32),
        ],
    )(zs_p, zf_p, att["W1"], pad8(att["b1"]), pad8(att["W2"]),
      dec["W1"], pad8(dec["b1"]), dec["W2"], pad8(dec["b2"]))


def _affine(su, sq, g, be):
    mean = su / N
    var = sq / N - mean * mean
    sc = g * lax.rsqrt(var + EPS)
    return sc, be - mean * sc


def kernel(x, shg, fhg, params):
    ps, pf = params["s"], params["f"]
    w1 = jnp.concatenate([ps["W1"], pf["W1"]], axis=1)
    hs, hf, st1 = _enc1(x, w1)
    sc_s, t_s = _affine(st1[0, :HID], st1[1, :HID], ps["g1"], ps["be1"])
    sc_f, t_f = _affine(st1[0, HID:], st1[1, HID:], pf["g1"], pf["be1"])

    vis = shg[0].reshape(NS * NK, K)
    eis = shg[1].reshape(NS * NK, K)
    vif = fhg[0].reshape(NS * NK, K)
    eif = fhg[1].reshape(NS * NK, K)
    vil = jnp.concatenate([vis, vif], axis=0)
    vig = jnp.concatenate([vis, vif + N], axis=0)
    eil = jnp.concatenate([eis, eif], axis=0)

    aff1 = jnp.stack([sc_s, t_s, sc_f, t_f])
    hcat = jnp.concatenate([hs, hf], axis=0)
    hq = HID // 4
    vo1_parts = _smooth_hid(
        *(hcat[:, i * hq:(i + 1) * hq] for i in range(4)),
        vil, vig, eil, aff1)
    vo1 = jnp.concatenate(vo1_parts, axis=1)
    vs_p = vo1[:N]
    vf_p = vo1[NP_PAD:NP_PAD + N]

    h2s, h2f, st2s, st2f = _enc2(vs_p, vf_p, ps["W2"], pf["W2"])
    sc2s, t2s = _affine(st2s[0], st2s[1], ps["g2"], ps["be2"])
    sc2f, t2f = _affine(st2f[0], st2f[1], pf["g2"], pf["be2"])
    aff2 = jnp.stack([sc2s, t2s, sc2f, t2f])
    h2cat = jnp.concatenate([h2s, h2f], axis=0)

    (vo2,) = _smooth_out(h2cat, vil, vig, eil, aff2)
    zs_p = vo2[:N]
    zf_p = vo2[NP_PAD:NP_PAD + N]

    z, zs, zf, xh = _fuse(zs_p, zf_p, params["att"], params["dec"])
    return (z, zs, zf, xh)


# speedup vs baseline: 1.1451x; 1.1451x over previous
import jax
import jax.numpy as jnp
from jax import lax
from jax.experimental import pallas as pl
from jax.experimental.pallas import tpu as pltpu
from jax.experimental.pallas import tpu_sc as plsc

N = 10000
E = 10000
P = 160000
IN_DIM = 256
HID = 128
OUT = 32
ATT_HID = 16
EPS = 1e-5

NS = 16
K = 80
NK = P // NS // K
assert NK % 2 == 1
NP_PAD = 10240
RN = NP_PAD // NS
RC = 128
NRC = RN // RC
BLK = 1000


def _make_smooth(D, do_relu, nh):
    Dh = D // nh
    CC = Dh // 16
    mesh = plsc.VectorSubcoreMesh(core_axis_name="c", subcore_axis_name="s")
    fdt = jnp.float32
    out_type = tuple(
        jax.ShapeDtypeStruct((2 * NP_PAD, Dh), fdt) for _ in range(nh)
    )
    scratch = [
        pltpu.VMEM((NK, K), jnp.int32),
        pltpu.VMEM((NK, K), jnp.int32),
        pltpu.VMEM((NK, K), jnp.int32),
        pltpu.VMEM((K, Dh), fdt),
        pltpu.VMEM((K, Dh), fdt),
        pltpu.VMEM((RC, Dh), fdt),
        pltpu.VMEM((RC, Dh), fdt),
        pltpu.VMEM((RN,), fdt),
        pltpu.VMEM((2, D), fdt),
        pltpu.VMEM((K,), fdt),
        pltpu.VMEM_SHARED((NP_PAD, Dh), fdt),
        pltpu.VMEM_SHARED((NP_PAD, Dh), fdt),
        pltpu.VMEM_SHARED((NP_PAD,), fdt),
        pltpu.VMEM_SHARED((NP_PAD,), fdt),
        pltpu.SemaphoreType.DMA,
        pltpu.SemaphoreType.DMA,
    ]

    def body(*refs):
        hparts = refs[:nh]
        vil_h, vig_h, eil_h, aff_h = refs[nh:nh + 4]
        vos = refs[nh + 4:nh + 4 + nh]
        (vil, vig, eil, rowsA, rowsB, nb, zb, cntb, stb, onesb,
         eacc, vacc, ecnt, vcnt, semA, semB) = refs[nh + 4 + nh:]

        def gsc_pipe(src, gi, scatter):
            pltpu.async_copy(src.at[gi.at[0]], rowsA, semA)

            def step(j, _):
                k = 2 * j
                pltpu.make_async_copy(src.at[gi.at[k]], rowsA, semA).wait()
                pltpu.async_copy(src.at[gi.at[k + 1]], rowsB, semB)
                scatter(rowsA, k)
                pltpu.make_async_copy(
                    src.at[gi.at[k + 1]], rowsB, semB).wait()
                pltpu.async_copy(src.at[gi.at[k + 2]], rowsA, semA)
                scatter(rowsB, k + 1)
                return 0

            lax.fori_loop(0, (NK - 1) // 2, step, 0)
            kl = NK - 1
            pltpu.make_async_copy(src.at[gi.at[kl]], rowsA, semA).wait()
            scatter(rowsA, kl)

        c = lax.axis_index("c")
        s = lax.axis_index("s")
        z16 = jnp.zeros((16,), fdt)
        o16 = jnp.ones((16,), fdt)
        r0 = s * RN
        ob0 = c * NP_PAD
        cb0 = c * (NS * NK) + s * NK

        def zb_fill(i, _):
            zb[i // CC, pl.ds((i % CC) * 16, 16)] = z16
            return 0
        lax.fori_loop(0, RC * CC, zb_fill, 0)

        def ones_fill(i, _):
            onesb[pl.ds(i * 16, 16)] = o16
            return 0
        lax.fori_loop(0, K // 16, ones_fill, 0)

        def cz_fill(i, _):
            cntb[pl.ds(i * 16, 16)] = z16
            return 0
        lax.fori_loop(0, RN // 16, cz_fill, 0)

        for j in range(NRC):
            pltpu.sync_copy(zb, eacc.at[pl.ds(r0 + j * RC, RC)])
            pltpu.sync_copy(zb, vacc.at[pl.ds(r0 + j * RC, RC)])
        pltpu.sync_copy(cntb, ecnt.at[pl.ds(r0, RN)])
        pltpu.sync_copy(cntb, vcnt.at[pl.ds(r0, RN)])

        pltpu.sync_copy(vil_h.at[pl.ds(cb0, NK)], vil)
        pltpu.sync_copy(vig_h.at[pl.ds(cb0, NK)], vig)
        pltpu.sync_copy(eil_h.at[pl.ds(cb0, NK)], eil)
        pltpu.sync_copy(aff_h.at[pl.ds(2 * c, 2)], stb)

        plsc.subcore_barrier()

        for h in range(nh):
            hcat = hparts[h]
            vo = vos[h]
            col0 = h * Dh

            if h == 0:
                def p1s(buf, k):
                    pltpu.sync_copy(buf, eacc.at[eil.at[k]], add=True)
                    pltpu.sync_copy(onesb, ecnt.at[eil.at[k]], add=True)
                    pltpu.sync_copy(onesb, vcnt.at[vil.at[k]], add=True)
            else:
                def p1s(buf, k):
                    pltpu.sync_copy(buf, eacc.at[eil.at[k]], add=True)
            gsc_pipe(hcat, vig, p1s)
            plsc.subcore_barrier()

            pltpu.sync_copy(ecnt.at[pl.ds(r0, RN)], cntb)
            for j in range(NRC):
                rb = r0 + j * RC
                pltpu.sync_copy(eacc.at[pl.ds(rb, RC)], nb)

                def enorm_grp(g, _):
                    cvg = cntb[pl.ds(j * RC + g * 16, 16)]
                    invg = 1.0 / jnp.maximum(cvg, 1.0)
                    tmg = jnp.where(cvg > 0.0, 1.0, 0.0)
                    iota = lax.iota(jnp.int32, 16)

                    def enorm_row(r2, _):
                        sel = iota == r2
                        inv = jnp.sum(jnp.where(sel, invg, 0.0))
                        tm = jnp.sum(jnp.where(sel, tmg, 0.0))
                        r = g * 16 + r2
                        for cc in range(CC):
                            v = nb[r, pl.ds(cc * 16, 16)]
                            sv = stb[0, pl.ds(col0 + cc * 16, 16)]
                            tv = stb[1, pl.ds(col0 + cc * 16, 16)]
                            nb[r, pl.ds(cc * 16, 16)] = v * sv * inv + tv * tm
                        return 0
                    lax.fori_loop(0, 16, enorm_row, 0)
                    return 0
                lax.fori_loop(0, RC // 16, enorm_grp, 0)

                pltpu.sync_copy(nb, eacc.at[pl.ds(rb, RC)])
            plsc.subcore_barrier()

            def p3s(buf, k):
                pltpu.sync_copy(buf, vacc.at[vil.at[k]], add=True)
            gsc_pipe(eacc, eil, p3s)
            plsc.subcore_barrier()

            pltpu.sync_copy(vcnt.at[pl.ds(r0, RN)], cntb)
            for j in range(NRC):
                rb = r0 + j * RC
                pltpu.sync_copy(vacc.at[pl.ds(rb, RC)], nb)
                if h < nh - 1:
                    pltpu.sync_copy(zb, vacc.at[pl.ds(rb, RC)])
                    pltpu.sync_copy(zb, eacc.at[pl.ds(rb, RC)])

                def vnorm_grp(g, _):
                    cvg = cntb[pl.ds(j * RC + g * 16, 16)]
                    invg = 1.0 / jnp.maximum(cvg, 1.0)
                    iota = lax.iota(jnp.int32, 16)

                    def vnorm_row(r2, _):
                        inv = jnp.sum(jnp.where(iota == r2, invg, 0.0))
                        r = g * 16 + r2
                        for cc in range(CC):
                            v = nb[r, pl.ds(cc * 16, 16)] * inv
                            if do_relu:
                                v = jnp.maximum(v, 0.0)
                            nb[r, pl.ds(cc * 16, 16)] = v
                        return 0
                    lax.fori_loop(0, 16, vnorm_row, 0)
                    return 0
                lax.fori_loop(0, RC // 16, vnorm_grp, 0)

                pltpu.sync_copy(nb, vo.at[pl.ds(ob0 + rb, RC)])
            if h < nh - 1:
                plsc.subcore_barrier()

    return pl.kernel(
        body, out_type=out_type, mesh=mesh, scratch_types=scratch,
        compiler_params=pltpu.CompilerParams(
            needs_layout_passes=False, use_tc_tiling_on_sc=False))


def _make_smooth_hbm(D, do_relu, nh):
    Dh = D // nh
    CC = Dh // 16
    mesh = plsc.VectorSubcoreMesh(core_axis_name="c", subcore_axis_name="s")
    fdt = jnp.float32
    out_type = tuple(
        jax.ShapeDtypeStruct((2 * NP_PAD, Dh), fdt) for _ in range(2 * nh)
    )
    scratch = [
        pltpu.VMEM((NK, K), jnp.int32),
        pltpu.VMEM((NK, K), jnp.int32),
        pltpu.VMEM((NK, K), jnp.int32),
        pltpu.VMEM((NK, K), jnp.int32),
        pltpu.VMEM((K, Dh), fdt),
        pltpu.VMEM((K, Dh), fdt),
        pltpu.VMEM((RC, Dh), fdt),
        pltpu.VMEM((RC, Dh), fdt),
        pltpu.VMEM((RN,), fdt),
        pltpu.VMEM((2, D), fdt),
        pltpu.VMEM((K,), fdt),
        pltpu.VMEM_SHARED((NP_PAD, Dh), fdt),
        pltpu.VMEM_SHARED((NP_PAD,), fdt),
        pltpu.VMEM_SHARED((NP_PAD,), fdt),
        pltpu.SemaphoreType.DMA,
        pltpu.SemaphoreType.DMA,
    ]

    def body(*refs):
        hparts = refs[:nh]
        vil_h, vig_h, eil_h, eig_h, aff_h = refs[nh:nh + 5]
        ens = refs[nh + 5:nh + 5 + nh]
        vos = refs[nh + 5 + nh:nh + 5 + 2 * nh]
        (vil, vig, eil, eig, rowsA, rowsB, nb, zb, cntb, stb, onesb,
         acc, ecnt, vcnt, semA, semB) = refs[nh + 5 + 2 * nh:]

        def gsc_pipe(src, gi, scatter):
            pltpu.async_copy(src.at[gi.at[0]], rowsA, semA)

            def step(j, _):
                k = 2 * j
                pltpu.make_async_copy(src.at[gi.at[k]], rowsA, semA).wait()
                pltpu.async_copy(src.at[gi.at[k + 1]], rowsB, semB)
                scatter(rowsA, k)
                pltpu.make_async_copy(
                    src.at[gi.at[k + 1]], rowsB, semB).wait()
                pltpu.async_copy(src.at[gi.at[k + 2]], rowsA, semA)
                scatter(rowsB, k + 1)
                return 0

            lax.fori_loop(0, (NK - 1) // 2, step, 0)
            kl = NK - 1
            pltpu.make_async_copy(src.at[gi.at[kl]], rowsA, semA).wait()
            scatter(rowsA, kl)

        c = lax.axis_index("c")
        s = lax.axis_index("s")
        z16 = jnp.zeros((16,), fdt)
        o16 = jnp.ones((16,), fdt)
        r0 = s * RN
        ob0 = c * NP_PAD
        cb0 = c * (NS * NK) + s * NK

        def zb_fill(i, _):
            zb[i // CC, pl.ds((i % CC) * 16, 16)] = z16
            return 0
        lax.fori_loop(0, RC * CC, zb_fill, 0)

        def ones_fill(i, _):
            onesb[pl.ds(i * 16, 16)] = o16
            return 0
        lax.fori_loop(0, K // 16, ones_fill, 0)

        def cz_fill(i, _):
            cntb[pl.ds(i * 16, 16)] = z16
            return 0
        lax.fori_loop(0, RN // 16, cz_fill, 0)

        for j in range(NRC):
            pltpu.sync_copy(zb, acc.at[pl.ds(r0 + j * RC, RC)])
        pltpu.sync_copy(cntb, ecnt.at[pl.ds(r0, RN)])
        pltpu.sync_copy(cntb, vcnt.at[pl.ds(r0, RN)])

        pltpu.sync_copy(vil_h.at[pl.ds(cb0, NK)], vil)
        pltpu.sync_copy(vig_h.at[pl.ds(cb0, NK)], vig)
        pltpu.sync_copy(eil_h.at[pl.ds(cb0, NK)], eil)
        pltpu.sync_copy(eig_h.at[pl.ds(cb0, NK)], eig)
        pltpu.sync_copy(aff_h.at[pl.ds(2 * c, 2)], stb)

        plsc.subcore_barrier()

        for h in range(nh):
            hcat = hparts[h]
            en = ens[h]
            vo = vos[h]
            col0 = h * Dh

            if h == 0:
                def p1s(buf, k):
                    pltpu.sync_copy(buf, acc.at[eil.at[k]], add=True)
                    pltpu.sync_copy(onesb, ecnt.at[eil.at[k]], add=True)
                    pltpu.sync_copy(onesb, vcnt.at[vil.at[k]], add=True)
            else:
                def p1s(buf, k):
                    pltpu.sync_copy(buf, acc.at[eil.at[k]], add=True)
            gsc_pipe(hcat, vig, p1s)
            plsc.subcore_barrier()

            pltpu.sync_copy(ecnt.at[pl.ds(r0, RN)], cntb)
            for j in range(NRC):
                rb = r0 + j * RC
                pltpu.sync_copy(acc.at[pl.ds(rb, RC)], nb)

                def enorm_grp(g, _):
                    cvg = cntb[pl.ds(j * RC + g * 16, 16)]
                    invg = 1.0 / jnp.maximum(cvg, 1.0)
                    tmg = jnp.where(cvg > 0.0, 1.0, 0.0)
                    iota = lax.iota(jnp.int32, 16)

                    def enorm_row(r2, _):
                        sel = iota == r2
                        inv = jnp.sum(jnp.where(sel, invg, 0.0))
                        tm = jnp.sum(jnp.where(sel, tmg, 0.0))
                        r = g * 16 + r2
                        for cc in range(CC):
                            v = nb[r, pl.ds(cc * 16, 16)]
                            sv = stb[0, pl.ds(col0 + cc * 16, 16)]
                            tv = stb[1, pl.ds(col0 + cc * 16, 16)]
                            nb[r, pl.ds(cc * 16, 16)] = v * sv * inv + tv * tm
                        return 0
                    lax.fori_loop(0, 16, enorm_row, 0)
                    return 0
                lax.fori_loop(0, RC // 16, enorm_grp, 0)

                pltpu.sync_copy(nb, en.at[pl.ds(ob0 + rb, RC)])
                pltpu.sync_copy(zb, acc.at[pl.ds(rb, RC)])
            plsc.subcore_barrier()

            def p3s(buf, k):
                pltpu.sync_copy(buf, acc.at[vil.at[k]], add=True)
            gsc_pipe(en, eig, p3s)
            plsc.subcore_barrier()

            pltpu.sync_copy(vcnt.at[pl.ds(r0, RN)], cntb)
            for j in range(NRC):
                rb = r0 + j * RC
                pltpu.sync_copy(acc.at[pl.ds(rb, RC)], nb)
                if h < nh - 1:
                    pltpu.sync_copy(zb, acc.at[pl.ds(rb, RC)])

                def vnorm_grp(g, _):
                    cvg = cntb[pl.ds(j * RC + g * 16, 16)]
                    invg = 1.0 / jnp.maximum(cvg, 1.0)
                    iota = lax.iota(jnp.int32, 16)

                    def vnorm_row(r2, _):
                        inv = jnp.sum(jnp.where(iota == r2, invg, 0.0))
                        r = g * 16 + r2
                        for cc in range(CC):
                            v = nb[r, pl.ds(cc * 16, 16)] * inv
                            if do_relu:
                                v = jnp.maximum(v, 0.0)
                            nb[r, pl.ds(cc * 16, 16)] = v
                        return 0
                    lax.fori_loop(0, 16, vnorm_row, 0)
                    return 0
                lax.fori_loop(0, RC // 16, vnorm_grp, 0)

                pltpu.sync_copy(nb, vo.at[pl.ds(ob0 + rb, RC)])
            if h < nh - 1:
                plsc.subcore_barrier()

    return pl.kernel(
        body, out_type=out_type, mesh=mesh, scratch_types=scratch,
        compiler_params=pltpu.CompilerParams(
            needs_layout_passes=False, use_tc_tiling_on_sc=False))


_smooth_hid = _make_smooth_hbm(HID, True, 2)
_smooth_out = _make_smooth(OUT, False, 1)


def _enc1_body(x_ref, w_ref, hs_ref, hf_ref, st_ref):
    i = pl.program_id(0)
    h = jnp.dot(x_ref[...], w_ref[...], preferred_element_type=jnp.float32)
    hs_ref[...] = h[:, :HID]
    hf_ref[...] = h[:, HID:]
    su = jnp.sum(h, axis=0, keepdims=True)
    sq = jnp.sum(h * h, axis=0, keepdims=True)
    st = jnp.concatenate([su, sq, jnp.zeros((6, 2 * HID), jnp.float32)], axis=0)

    @pl.when(i == 0)
    def _():
        st_ref[...] = jnp.zeros_like(st_ref)

    st_ref[...] += st


def _enc1(x, w):
    return pl.pallas_call(
        _enc1_body,
        grid=(N // BLK,),
        in_specs=[
            pl.BlockSpec((BLK, IN_DIM), lambda i: (i, 0)),
            pl.BlockSpec((IN_DIM, 2 * HID), lambda i: (0, 0)),
        ],
        out_specs=[
            pl.BlockSpec((BLK, HID), lambda i: (i, 0)),
            pl.BlockSpec((BLK, HID), lambda i: (i, 0)),
            pl.BlockSpec((8, 2 * HID), lambda i: (0, 0)),
        ],
        out_shape=[
            jax.ShapeDtypeStruct((N, HID), jnp.float32),
            jax.ShapeDtypeStruct((N, HID), jnp.float32),
            jax.ShapeDtypeStruct((8, 2 * HID), jnp.float32),
        ],
    )(x, w)


def _enc2_body(vs_ref, vf_ref, ws_ref, wf_ref, hs_ref, hf_ref, sts_ref, stf_ref):
    i = pl.program_id(0)
    h2s = jnp.dot(vs_ref[...], ws_ref[...], preferred_element_type=jnp.float32)
    h2f = jnp.dot(vf_ref[...], wf_ref[...], preferred_element_type=jnp.float32)
    hs_ref[...] = h2s
    hf_ref[...] = h2f
    for h2, st_ref in ((h2s, sts_ref), (h2f, stf_ref)):
        su = jnp.sum(h2, axis=0, keepdims=True)
        sq = jnp.sum(h2 * h2, axis=0, keepdims=True)
        st = jnp.concatenate([su, sq, jnp.zeros((6, OUT), jnp.float32)], axis=0)

        @pl.when(i == 0)
        def _():
            st_ref[...] = jnp.zeros_like(st_ref)

        st_ref[...] += st


def _enc2(vs_p, vf_p, w2s, w2f):
    return pl.pallas_call(
        _enc2_body,
        grid=(N // BLK,),
        in_specs=[
            pl.BlockSpec((BLK, HID), lambda i: (i, 0)),
            pl.BlockSpec((BLK, HID), lambda i: (i, 0)),
            pl.BlockSpec((HID, OUT), lambda i: (0, 0)),
            pl.BlockSpec((HID, OUT), lambda i: (0, 0)),
        ],
        out_specs=[
            pl.BlockSpec((BLK, OUT), lambda i: (i, 0)),
            pl.BlockSpec((BLK, OUT), lambda i: (i, 0)),
            pl.BlockSpec((8, OUT), lambda i: (0, 0)),
            pl.BlockSpec((8, OUT), lambda i: (0, 0)),
        ],
        out_shape=[
            jax.ShapeDtypeStruct((N, OUT), jnp.float32),
            jax.ShapeDtypeStruct((N, OUT), jnp.float32),
            jax.ShapeDtypeStruct((8, OUT), jnp.float32),
            jax.ShapeDtypeStruct((8, OUT), jnp.float32),
        ],
    )(vs_p, vf_p, w2s, w2f)


def _fuse_body(zs_ref, zf_ref, aw1_ref, ab1_ref, aw2_ref,
               dw1_ref, db1_ref, dw2_ref, db2_ref,
               z_ref, zso_ref, zfo_ref, xh_ref):
    zs = zs_ref[...]
    zf = zf_ref[...]
    ab1 = ab1_ref[0:1, :]
    ts = jnp.tanh(jnp.dot(zs, aw1_ref[...], preferred_element_type=jnp.float32) + ab1)
    tf = jnp.tanh(jnp.dot(zf, aw1_ref[...], preferred_element_type=jnp.float32) + ab1)
    aw2 = aw2_ref[0:1, :]
    ws = jnp.sum(ts * aw2, axis=1, keepdims=True)
    wf = jnp.sum(tf * aw2, axis=1, keepdims=True)
    m = jnp.maximum(ws, wf)
    es = jnp.exp(ws - m)
    ef = jnp.exp(wf - m)
    den = es + ef
    z = (es / den) * zs + (ef / den) * zf
    z_ref[...] = z
    zso_ref[...] = zs
    zfo_ref[...] = zf
    hd = jnp.maximum(
        jnp.dot(z, dw1_ref[...], preferred_element_type=jnp.float32) + db1_ref[0:1, :], 0.0)
    xh_ref[...] = jnp.dot(hd, dw2_ref[...], preferred_element_type=jnp.float32) + db2_ref[0:1, :]


def _fuse(zs_p, zf_p, att, dec):
    def pad8(v):
        return jnp.broadcast_to(v.reshape(1, -1), (8, v.size))

    return pl.pallas_call(
        _fuse_body,
        grid=(N // BLK,),
        in_specs=[
            pl.BlockSpec((BLK, OUT), lambda i: (i, 0)),
            pl.BlockSpec((BLK, OUT), lambda i: (i, 0)),
            pl.BlockSpec((OUT, ATT_HID), lambda i: (0, 0)),
            pl.BlockSpec((8, ATT_HID), lambda i: (0, 0)),
            pl.BlockSpec((8, ATT_HID), lambda i: (0, 0)),
            pl.BlockSpec((OUT, HID), lambda i: (0, 0)),
            pl.BlockSpec((8, HID), lambda i: (0, 0)),
            pl.BlockSpec((HID, IN_DIM), lambda i: (0, 0)),
            pl.BlockSpec((8, IN_DIM), lambda i: (0, 0)),
        ],
        out_specs=[
            pl.BlockSpec((BLK, OUT), lambda i: (i, 0)),
            pl.BlockSpec((BLK, OUT), lambda i: (i, 0)),
            pl.BlockSpec((BLK, OUT), lambda i: (i, 0)),
            pl.BlockSpec((BLK, IN_DIM), lambda i: (i, 0)),
        ],
        out_shape=[
            jax.ShapeDtypeStruct((N, OUT), jnp.float32),
            jax.ShapeDtypeStruct((N, OUT), jnp.float32),
            jax.ShapeDtypeStruct((N, OUT), jnp.float32),
            jax.ShapeDtypeStruct((N, IN_DIM), jnp.float32),
        ],
    )(zs_p, zf_p, att["W1"], pad8(att["b1"]), pad8(att["W2"]),
      dec["W1"], pad8(dec["b1"]), dec["W2"], pad8(dec["b2"]))


def _affine(su, sq, g, be):
    mean = su / N
    var = sq / N - mean * mean
    sc = g * lax.rsqrt(var + EPS)
    return sc, be - mean * sc


def kernel(x, shg, fhg, params):
    ps, pf = params["s"], params["f"]
    w1 = jnp.concatenate([ps["W1"], pf["W1"]], axis=1)
    hs, hf, st1 = _enc1(x, w1)
    sc_s, t_s = _affine(st1[0, :HID], st1[1, :HID], ps["g1"], ps["be1"])
    sc_f, t_f = _affine(st1[0, HID:], st1[1, HID:], pf["g1"], pf["be1"])

    vis = shg[0].reshape(NS * NK, K)
    eis = shg[1].reshape(NS * NK, K)
    vif = fhg[0].reshape(NS * NK, K)
    eif = fhg[1].reshape(NS * NK, K)
    vil = jnp.concatenate([vis, vif], axis=0)
    vig = jnp.concatenate([vis, vif + N], axis=0)
    eil = jnp.concatenate([eis, eif], axis=0)
    eig = jnp.concatenate([eis, eif + NP_PAD], axis=0)

    aff1 = jnp.stack([sc_s, t_s, sc_f, t_f])
    hcat = jnp.concatenate([hs, hf], axis=0)
    hd2 = HID // 2
    _, _, vo1a, vo1b = _smooth_hid(
        hcat[:, :hd2], hcat[:, hd2:], vil, vig, eil, eig, aff1)
    vo1 = jnp.concatenate([vo1a, vo1b], axis=1)
    vs_p = vo1[:N]
    vf_p = vo1[NP_PAD:NP_PAD + N]

    h2s, h2f, st2s, st2f = _enc2(vs_p, vf_p, ps["W2"], pf["W2"])
    sc2s, t2s = _affine(st2s[0], st2s[1], ps["g2"], ps["be2"])
    sc2f, t2f = _affine(st2f[0], st2f[1], pf["g2"], pf["be2"])
    aff2 = jnp.stack([sc2s, t2s, sc2f, t2f])
    h2cat = jnp.concatenate([h2s, h2f], axis=0)

    (vo2,) = _smooth_out(h2cat, vil, vig, eil, aff2)
    zs_p = vo2[:N]
    zf_p = vo2[NP_PAD:NP_PAD + N]

    z, zs, zf, xh = _fuse(zs_p, zf_p, params["att"], params["dec"])
    return (z, zs, zf, xh)


# speedup vs baseline: 1.3790x; 1.2042x over previous
import jax
import jax.numpy as jnp
from jax import lax
from jax.experimental import pallas as pl
from jax.experimental.pallas import tpu as pltpu
from jax.experimental.pallas import tpu_sc as plsc

N = 10000
E = 10000
P = 160000
IN_DIM = 256
HID = 128
OUT = 32
ATT_HID = 16
EPS = 1e-5

NS = 16
K = 125
KP = 128
NK = P // NS // K
assert NK * K * NS == P
NP_PAD = 10240
RN = NP_PAD // NS
RC = 128
NRC = RN // RC
BLK = 1000


def _make_smooth(D, do_relu, nh):
    Dh = D // nh
    CC = Dh // 16
    mesh = plsc.VectorSubcoreMesh(core_axis_name="c", subcore_axis_name="s")
    fdt = jnp.float32
    out_type = tuple(
        jax.ShapeDtypeStruct((2 * NP_PAD, Dh), fdt) for _ in range(nh)
    )
    scratch = [
        pltpu.VMEM((NK, K), jnp.int32),
        pltpu.VMEM((NK, K), jnp.int32),
        pltpu.VMEM((NK, K), jnp.int32),
        pltpu.VMEM((K, Dh), fdt),
        pltpu.VMEM((K, Dh), fdt),
        pltpu.VMEM((RC, Dh), fdt),
        pltpu.VMEM((RC, Dh), fdt),
        pltpu.VMEM((RN,), fdt),
        pltpu.VMEM((2, D), fdt),
        pltpu.VMEM((KP,), fdt),
        pltpu.VMEM_SHARED((NP_PAD, Dh), fdt),
        pltpu.VMEM_SHARED((NP_PAD, Dh), fdt),
        pltpu.VMEM_SHARED((NP_PAD,), fdt),
        pltpu.VMEM_SHARED((NP_PAD,), fdt),
        pltpu.SemaphoreType.DMA,
        pltpu.SemaphoreType.DMA,
    ]

    def body(*refs):
        hparts = refs[:nh]
        vil_h, vig_h, eil_h, aff_h = refs[nh:nh + 4]
        vos = refs[nh + 4:nh + 4 + nh]
        (vil, vig, eil, rowsA, rowsB, nb, zb, cntb, stb, onesb,
         eacc, vacc, ecnt, vcnt, semA, semB) = refs[nh + 4 + nh:]

        def gsc_pipe(src, gi, scatter):
            pltpu.async_copy(src.at[gi.at[0]], rowsA, semA)

            def step(j, _):
                k = 2 * j
                pltpu.make_async_copy(src.at[gi.at[k]], rowsA, semA).wait()
                pltpu.async_copy(src.at[gi.at[k + 1]], rowsB, semB)
                scatter(rowsA, k)
                pltpu.make_async_copy(
                    src.at[gi.at[k + 1]], rowsB, semB).wait()
                pltpu.async_copy(src.at[gi.at[k + 2]], rowsA, semA)
                scatter(rowsB, k + 1)
                return 0

            lax.fori_loop(0, (NK - 1) // 2, step, 0)
            if NK % 2:
                kl = NK - 1
                pltpu.make_async_copy(src.at[gi.at[kl]], rowsA, semA).wait()
                scatter(rowsA, kl)
            else:
                k0 = NK - 2
                pltpu.make_async_copy(src.at[gi.at[k0]], rowsA, semA).wait()
                pltpu.async_copy(src.at[gi.at[k0 + 1]], rowsB, semB)
                scatter(rowsA, k0)
                pltpu.make_async_copy(
                    src.at[gi.at[k0 + 1]], rowsB, semB).wait()
                scatter(rowsB, k0 + 1)

        c = lax.axis_index("c")
        s = lax.axis_index("s")
        z16 = jnp.zeros((16,), fdt)
        o16 = jnp.ones((16,), fdt)
        r0 = s * RN
        ob0 = c * NP_PAD
        cb0 = c * (NS * NK) + s * NK

        def zb_fill(i, _):
            zb[i // CC, pl.ds((i % CC) * 16, 16)] = z16
            return 0
        lax.fori_loop(0, RC * CC, zb_fill, 0)

        def ones_fill(i, _):
            onesb[pl.ds(i * 16, 16)] = o16
            return 0
        lax.fori_loop(0, KP // 16, ones_fill, 0)

        def cz_fill(i, _):
            cntb[pl.ds(i * 16, 16)] = z16
            return 0
        lax.fori_loop(0, RN // 16, cz_fill, 0)

        for j in range(NRC):
            pltpu.sync_copy(zb, eacc.at[pl.ds(r0 + j * RC, RC)])
            pltpu.sync_copy(zb, vacc.at[pl.ds(r0 + j * RC, RC)])
        pltpu.sync_copy(cntb, ecnt.at[pl.ds(r0, RN)])
        pltpu.sync_copy(cntb, vcnt.at[pl.ds(r0, RN)])

        pltpu.sync_copy(vil_h.at[pl.ds(cb0, NK)], vil)
        pltpu.sync_copy(vig_h.at[pl.ds(cb0, NK)], vig)
        pltpu.sync_copy(eil_h.at[pl.ds(cb0, NK)], eil)
        pltpu.sync_copy(aff_h.at[pl.ds(2 * c, 2)], stb)

        plsc.subcore_barrier()

        for h in range(nh):
            hcat = hparts[h]
            vo = vos[h]
            col0 = h * Dh

            if h == 0:
                def p1s(buf, k):
                    pltpu.sync_copy(buf, eacc.at[eil.at[k]], add=True)
                    pltpu.sync_copy(
                        onesb.at[pl.ds(0, K)], ecnt.at[eil.at[k]], add=True)
                    pltpu.sync_copy(
                        onesb.at[pl.ds(0, K)], vcnt.at[vil.at[k]], add=True)
            else:
                def p1s(buf, k):
                    pltpu.sync_copy(buf, eacc.at[eil.at[k]], add=True)
            gsc_pipe(hcat, vig, p1s)
            plsc.subcore_barrier()

            pltpu.sync_copy(ecnt.at[pl.ds(r0, RN)], cntb)
            for j in range(NRC):
                rb = r0 + j * RC
                pltpu.sync_copy(eacc.at[pl.ds(rb, RC)], nb)

                def enorm_grp(g, _):
                    cvg = cntb[pl.ds(j * RC + g * 16, 16)]
                    invg = 1.0 / jnp.maximum(cvg, 1.0)
                    tmg = jnp.where(cvg > 0.0, 1.0, 0.0)
                    iota = lax.iota(jnp.int32, 16)

                    def enorm_row(r2, _):
                        sel = iota == r2
                        inv = jnp.sum(jnp.where(sel, invg, 0.0))
                        tm = jnp.sum(jnp.where(sel, tmg, 0.0))
                        r = g * 16 + r2
                        for cc in range(CC):
                            v = nb[r, pl.ds(cc * 16, 16)]
                            sv = stb[0, pl.ds(col0 + cc * 16, 16)]
                            tv = stb[1, pl.ds(col0 + cc * 16, 16)]
                            nb[r, pl.ds(cc * 16, 16)] = v * sv * inv + tv * tm
                        return 0
                    lax.fori_loop(0, 16, enorm_row, 0)
                    return 0
                lax.fori_loop(0, RC // 16, enorm_grp, 0)

                pltpu.sync_copy(nb, eacc.at[pl.ds(rb, RC)])
            plsc.subcore_barrier()

            def p3s(buf, k):
                pltpu.sync_copy(buf, vacc.at[vil.at[k]], add=True)
            gsc_pipe(eacc, eil, p3s)
            plsc.subcore_barrier()

            pltpu.sync_copy(vcnt.at[pl.ds(r0, RN)], cntb)
            for j in range(NRC):
                rb = r0 + j * RC
                pltpu.sync_copy(vacc.at[pl.ds(rb, RC)], nb)
                if h < nh - 1:
                    pltpu.sync_copy(zb, vacc.at[pl.ds(rb, RC)])
                    pltpu.sync_copy(zb, eacc.at[pl.ds(rb, RC)])

                def vnorm_grp(g, _):
                    cvg = cntb[pl.ds(j * RC + g * 16, 16)]
                    invg = 1.0 / jnp.maximum(cvg, 1.0)
                    iota = lax.iota(jnp.int32, 16)

                    def vnorm_row(r2, _):
                        inv = jnp.sum(jnp.where(iota == r2, invg, 0.0))
                        r = g * 16 + r2
                        for cc in range(CC):
                            v = nb[r, pl.ds(cc * 16, 16)] * inv
                            if do_relu:
                                v = jnp.maximum(v, 0.0)
                            nb[r, pl.ds(cc * 16, 16)] = v
                        return 0
                    lax.fori_loop(0, 16, vnorm_row, 0)
                    return 0
                lax.fori_loop(0, RC // 16, vnorm_grp, 0)

                pltpu.sync_copy(nb, vo.at[pl.ds(ob0 + rb, RC)])
            if h < nh - 1:
                plsc.subcore_barrier()

    return pl.kernel(
        body, out_type=out_type, mesh=mesh, scratch_types=scratch,
        compiler_params=pltpu.CompilerParams(
            needs_layout_passes=False, use_tc_tiling_on_sc=False))


def _make_smooth_hbm(D, do_relu, nh):
    Dh = D // nh
    CC = Dh // 16
    mesh = plsc.VectorSubcoreMesh(core_axis_name="c", subcore_axis_name="s")
    fdt = jnp.float32
    out_type = tuple(
        jax.ShapeDtypeStruct((2 * NP_PAD, Dh), fdt) for _ in range(2 * nh)
    )
    scratch = [
        pltpu.VMEM((NK, K), jnp.int32),
        pltpu.VMEM((NK, K), jnp.int32),
        pltpu.VMEM((NK, K), jnp.int32),
        pltpu.VMEM((NK, K), jnp.int32),
        pltpu.VMEM((K, Dh), fdt),
        pltpu.VMEM((K, Dh), fdt),
        pltpu.VMEM((RC, Dh), fdt),
        pltpu.VMEM((RC, Dh), fdt),
        pltpu.VMEM((RN,), fdt),
        pltpu.VMEM((2, D), fdt),
        pltpu.VMEM((KP,), fdt),
        pltpu.VMEM_SHARED((NP_PAD, Dh), fdt),
        pltpu.VMEM_SHARED((NP_PAD,), fdt),
        pltpu.VMEM_SHARED((NP_PAD,), fdt),
        pltpu.SemaphoreType.DMA,
        pltpu.SemaphoreType.DMA,
    ]

    def body(*refs):
        hparts = refs[:nh]
        vil_h, vig_h, eil_h, eig_h, aff_h = refs[nh:nh + 5]
        ens = refs[nh + 5:nh + 5 + nh]
        vos = refs[nh + 5 + nh:nh + 5 + 2 * nh]
        (vil, vig, eil, eig, rowsA, rowsB, nb, zb, cntb, stb, onesb,
         acc, ecnt, vcnt, semA, semB) = refs[nh + 5 + 2 * nh:]

        def gsc_pipe(src, gi, scatter):
            pltpu.async_copy(src.at[gi.at[0]], rowsA, semA)

            def step(j, _):
                k = 2 * j
                pltpu.make_async_copy(src.at[gi.at[k]], rowsA, semA).wait()
                pltpu.async_copy(src.at[gi.at[k + 1]], rowsB, semB)
                scatter(rowsA, k)
                pltpu.make_async_copy(
                    src.at[gi.at[k + 1]], rowsB, semB).wait()
                pltpu.async_copy(src.at[gi.at[k + 2]], rowsA, semA)
                scatter(rowsB, k + 1)
                return 0

            lax.fori_loop(0, (NK - 1) // 2, step, 0)
            if NK % 2:
                kl = NK - 1
                pltpu.make_async_copy(src.at[gi.at[kl]], rowsA, semA).wait()
                scatter(rowsA, kl)
            else:
                k0 = NK - 2
                pltpu.make_async_copy(src.at[gi.at[k0]], rowsA, semA).wait()
                pltpu.async_copy(src.at[gi.at[k0 + 1]], rowsB, semB)
                scatter(rowsA, k0)
                pltpu.make_async_copy(
                    src.at[gi.at[k0 + 1]], rowsB, semB).wait()
                scatter(rowsB, k0 + 1)

        c = lax.axis_index("c")
        s = lax.axis_index("s")
        z16 = jnp.zeros((16,), fdt)
        o16 = jnp.ones((16,), fdt)
        r0 = s * RN
        ob0 = c * NP_PAD
        cb0 = c * (NS * NK) + s * NK

        def zb_fill(i, _):
            zb[i // CC, pl.ds((i % CC) * 16, 16)] = z16
            return 0
        lax.fori_loop(0, RC * CC, zb_fill, 0)

        def ones_fill(i, _):
            onesb[pl.ds(i * 16, 16)] = o16
            return 0
        lax.fori_loop(0, KP // 16, ones_fill, 0)

        def cz_fill(i, _):
            cntb[pl.ds(i * 16, 16)] = z16
            return 0
        lax.fori_loop(0, RN // 16, cz_fill, 0)

        for j in range(NRC):
            pltpu.sync_copy(zb, acc.at[pl.ds(r0 + j * RC, RC)])
        pltpu.sync_copy(cntb, ecnt.at[pl.ds(r0, RN)])
        pltpu.sync_copy(cntb, vcnt.at[pl.ds(r0, RN)])

        pltpu.sync_copy(vil_h.at[pl.ds(cb0, NK)], vil)
        pltpu.sync_copy(vig_h.at[pl.ds(cb0, NK)], vig)
        pltpu.sync_copy(eil_h.at[pl.ds(cb0, NK)], eil)
        pltpu.sync_copy(eig_h.at[pl.ds(cb0, NK)], eig)
        pltpu.sync_copy(aff_h.at[pl.ds(2 * c, 2)], stb)

        plsc.subcore_barrier()

        for h in range(nh):
            hcat = hparts[h]
            en = ens[h]
            vo = vos[h]
            col0 = h * Dh

            if h == 0:
                def p1s(buf, k):
                    pltpu.sync_copy(buf, acc.at[eil.at[k]], add=True)
                    pltpu.sync_copy(
                        onesb.at[pl.ds(0, K)], ecnt.at[eil.at[k]], add=True)
                    pltpu.sync_copy(
                        onesb.at[pl.ds(0, K)], vcnt.at[vil.at[k]], add=True)
            else:
                def p1s(buf, k):
                    pltpu.sync_copy(buf, acc.at[eil.at[k]], add=True)
            gsc_pipe(hcat, vig, p1s)
            plsc.subcore_barrier()

            pltpu.sync_copy(ecnt.at[pl.ds(r0, RN)], cntb)
            for j in range(NRC):
                rb = r0 + j * RC
                pltpu.sync_copy(acc.at[pl.ds(rb, RC)], nb)

                def enorm_grp(g, _):
                    cvg = cntb[pl.ds(j * RC + g * 16, 16)]
                    invg = 1.0 / jnp.maximum(cvg, 1.0)
                    tmg = jnp.where(cvg > 0.0, 1.0, 0.0)
                    iota = lax.iota(jnp.int32, 16)

                    def enorm_row(r2, _):
                        sel = iota == r2
                        inv = jnp.sum(jnp.where(sel, invg, 0.0))
                        tm = jnp.sum(jnp.where(sel, tmg, 0.0))
                        r = g * 16 + r2
                        for cc in range(CC):
                            v = nb[r, pl.ds(cc * 16, 16)]
                            sv = stb[0, pl.ds(col0 + cc * 16, 16)]
                            tv = stb[1, pl.ds(col0 + cc * 16, 16)]
                            nb[r, pl.ds(cc * 16, 16)] = v * sv * inv + tv * tm
                        return 0
                    lax.fori_loop(0, 16, enorm_row, 0)
                    return 0
                lax.fori_loop(0, RC // 16, enorm_grp, 0)

                pltpu.sync_copy(nb, en.at[pl.ds(ob0 + rb, RC)])
                pltpu.sync_copy(zb, acc.at[pl.ds(rb, RC)])
            plsc.subcore_barrier()

            def p3s(buf, k):
                pltpu.sync_copy(buf, acc.at[vil.at[k]], add=True)
            gsc_pipe(en, eig, p3s)
            plsc.subcore_barrier()

            pltpu.sync_copy(vcnt.at[pl.ds(r0, RN)], cntb)
            for j in range(NRC):
                rb = r0 + j * RC
                pltpu.sync_copy(acc.at[pl.ds(rb, RC)], nb)
                if h < nh - 1:
                    pltpu.sync_copy(zb, acc.at[pl.ds(rb, RC)])

                def vnorm_grp(g, _):
                    cvg = cntb[pl.ds(j * RC + g * 16, 16)]
                    invg = 1.0 / jnp.maximum(cvg, 1.0)
                    iota = lax.iota(jnp.int32, 16)

                    def vnorm_row(r2, _):
                        inv = jnp.sum(jnp.where(iota == r2, invg, 0.0))
                        r = g * 16 + r2
                        for cc in range(CC):
                            v = nb[r, pl.ds(cc * 16, 16)] * inv
                            if do_relu:
                                v = jnp.maximum(v, 0.0)
                            nb[r, pl.ds(cc * 16, 16)] = v
                        return 0
                    lax.fori_loop(0, 16, vnorm_row, 0)
                    return 0
                lax.fori_loop(0, RC // 16, vnorm_grp, 0)

                pltpu.sync_copy(nb, vo.at[pl.ds(ob0 + rb, RC)])
            if h < nh - 1:
                plsc.subcore_barrier()

    return pl.kernel(
        body, out_type=out_type, mesh=mesh, scratch_types=scratch,
        compiler_params=pltpu.CompilerParams(
            needs_layout_passes=False, use_tc_tiling_on_sc=False))


_smooth_hid = _make_smooth_hbm(HID, True, 2)
_smooth_out = _make_smooth(OUT, False, 1)


def _enc1_body(x_ref, w_ref, hs_ref, hf_ref, st_ref):
    i = pl.program_id(0)
    h = jnp.dot(x_ref[...], w_ref[...], preferred_element_type=jnp.float32)
    hs_ref[...] = h[:, :HID]
    hf_ref[...] = h[:, HID:]
    su = jnp.sum(h, axis=0, keepdims=True)
    sq = jnp.sum(h * h, axis=0, keepdims=True)
    st = jnp.concatenate([su, sq, jnp.zeros((6, 2 * HID), jnp.float32)], axis=0)

    @pl.when(i == 0)
    def _():
        st_ref[...] = jnp.zeros_like(st_ref)

    st_ref[...] += st


def _enc1(x, w):
    return pl.pallas_call(
        _enc1_body,
        grid=(N // BLK,),
        in_specs=[
            pl.BlockSpec((BLK, IN_DIM), lambda i: (i, 0)),
            pl.BlockSpec((IN_DIM, 2 * HID), lambda i: (0, 0)),
        ],
        out_specs=[
            pl.BlockSpec((BLK, HID), lambda i: (i, 0)),
            pl.BlockSpec((BLK, HID), lambda i: (i, 0)),
            pl.BlockSpec((8, 2 * HID), lambda i: (0, 0)),
        ],
        out_shape=[
            jax.ShapeDtypeStruct((N, HID), jnp.float32),
            jax.ShapeDtypeStruct((N, HID), jnp.float32),
            jax.ShapeDtypeStruct((8, 2 * HID), jnp.float32),
        ],
    )(x, w)


def _enc2_body(vs_ref, vf_ref, ws_ref, wf_ref, hs_ref, hf_ref, sts_ref, stf_ref):
    i = pl.program_id(0)
    h2s = jnp.dot(vs_ref[...], ws_ref[...], preferred_element_type=jnp.float32)
    h2f = jnp.dot(vf_ref[...], wf_ref[...], preferred_element_type=jnp.float32)
    hs_ref[...] = h2s
    hf_ref[...] = h2f
    for h2, st_ref in ((h2s, sts_ref), (h2f, stf_ref)):
        su = jnp.sum(h2, axis=0, keepdims=True)
        sq = jnp.sum(h2 * h2, axis=0, keepdims=True)
        st = jnp.concatenate([su, sq, jnp.zeros((6, OUT), jnp.float32)], axis=0)

        @pl.when(i == 0)
        def _():
            st_ref[...] = jnp.zeros_like(st_ref)

        st_ref[...] += st


def _enc2(vs_p, vf_p, w2s, w2f):
    return pl.pallas_call(
        _enc2_body,
        grid=(N // BLK,),
        in_specs=[
            pl.BlockSpec((BLK, HID), lambda i: (i, 0)),
            pl.BlockSpec((BLK, HID), lambda i: (i, 0)),
            pl.BlockSpec((HID, OUT), lambda i: (0, 0)),
            pl.BlockSpec((HID, OUT), lambda i: (0, 0)),
        ],
        out_specs=[
            pl.BlockSpec((BLK, OUT), lambda i: (i, 0)),
            pl.BlockSpec((BLK, OUT), lambda i: (i, 0)),
            pl.BlockSpec((8, OUT), lambda i: (0, 0)),
            pl.BlockSpec((8, OUT), lambda i: (0, 0)),
        ],
        out_shape=[
            jax.ShapeDtypeStruct((N, OUT), jnp.float32),
            jax.ShapeDtypeStruct((N, OUT), jnp.float32),
            jax.ShapeDtypeStruct((8, OUT), jnp.float32),
            jax.ShapeDtypeStruct((8, OUT), jnp.float32),
        ],
    )(vs_p, vf_p, w2s, w2f)


def _fuse_body(zs_ref, zf_ref, aw1_ref, ab1_ref, aw2_ref,
               dw1_ref, db1_ref, dw2_ref, db2_ref,
               z_ref, zso_ref, zfo_ref, xh_ref):
    zs = zs_ref[...]
    zf = zf_ref[...]
    ab1 = ab1_ref[0:1, :]
    ts = jnp.tanh(jnp.dot(zs, aw1_ref[...], preferred_element_type=jnp.float32) + ab1)
    tf = jnp.tanh(jnp.dot(zf, aw1_ref[...], preferred_element_type=jnp.float32) + ab1)
    aw2 = aw2_ref[0:1, :]
    ws = jnp.sum(ts * aw2, axis=1, keepdims=True)
    wf = jnp.sum(tf * aw2, axis=1, keepdims=True)
    m = jnp.maximum(ws, wf)
    es = jnp.exp(ws - m)
    ef = jnp.exp(wf - m)
    den = es + ef
    z = (es / den) * zs + (ef / den) * zf
    z_ref[...] = z
    zso_ref[...] = zs
    zfo_ref[...] = zf
    hd = jnp.maximum(
        jnp.dot(z, dw1_ref[...], preferred_element_type=jnp.float32) + db1_ref[0:1, :], 0.0)
    xh_ref[...] = jnp.dot(hd, dw2_ref[...], preferred_element_type=jnp.float32) + db2_ref[0:1, :]


def _fuse(zs_p, zf_p, att, dec):
    def pad8(v):
        return jnp.broadcast_to(v.reshape(1, -1), (8, v.size))

    return pl.pallas_call(
        _fuse_body,
        grid=(N // BLK,),
        in_specs=[
            pl.BlockSpec((BLK, OUT), lambda i: (i, 0)),
            pl.BlockSpec((BLK, OUT), lambda i: (i, 0)),
            pl.BlockSpec((OUT, ATT_HID), lambda i: (0, 0)),
            pl.BlockSpec((8, ATT_HID), lambda i: (0, 0)),
            pl.BlockSpec((8, ATT_HID), lambda i: (0, 0)),
            pl.BlockSpec((OUT, HID), lambda i: (0, 0)),
            pl.BlockSpec((8, HID), lambda i: (0, 0)),
            pl.BlockSpec((HID, IN_DIM), lambda i: (0, 0)),
            pl.BlockSpec((8, IN_DIM), lambda i: (0, 0)),
        ],
        out_specs=[
            pl.BlockSpec((BLK, OUT), lambda i: (i, 0)),
            pl.BlockSpec((BLK, OUT), lambda i: (i, 0)),
            pl.BlockSpec((BLK, OUT), lambda i: (i, 0)),
            pl.BlockSpec((BLK, IN_DIM), lambda i: (i, 0)),
        ],
        out_shape=[
            jax.ShapeDtypeStruct((N, OUT), jnp.float32),
            jax.ShapeDtypeStruct((N, OUT), jnp.float32),
            jax.ShapeDtypeStruct((N, OUT), jnp.float32),
            jax.ShapeDtypeStruct((N, IN_DIM), jnp.float32),
        ],
    )(zs_p, zf_p, att["W1"], pad8(att["b1"]), pad8(att["W2"]),
      dec["W1"], pad8(dec["b1"]), dec["W2"], pad8(dec["b2"]))


def _affine(su, sq, g, be):
    mean = su / N
    var = sq / N - mean * mean
    sc = g * lax.rsqrt(var + EPS)
    return sc, be - mean * sc


def kernel(x, shg, fhg, params):
    ps, pf = params["s"], params["f"]
    w1 = jnp.concatenate([ps["W1"], pf["W1"]], axis=1)
    hs, hf, st1 = _enc1(x, w1)
    sc_s, t_s = _affine(st1[0, :HID], st1[1, :HID], ps["g1"], ps["be1"])
    sc_f, t_f = _affine(st1[0, HID:], st1[1, HID:], pf["g1"], pf["be1"])

    vis = shg[0].reshape(NS * NK, K)
    eis = shg[1].reshape(NS * NK, K)
    vif = fhg[0].reshape(NS * NK, K)
    eif = fhg[1].reshape(NS * NK, K)
    vil = jnp.concatenate([vis, vif], axis=0)
    vig = jnp.concatenate([vis, vif + N], axis=0)
    eil = jnp.concatenate([eis, eif], axis=0)
    eig = jnp.concatenate([eis, eif + NP_PAD], axis=0)

    aff1 = jnp.stack([sc_s, t_s, sc_f, t_f])
    hcat = jnp.concatenate([hs, hf], axis=0)
    hd2 = HID // 2
    _, _, vo1a, vo1b = _smooth_hid(
        hcat[:, :hd2], hcat[:, hd2:], vil, vig, eil, eig, aff1)
    vo1 = jnp.concatenate([vo1a, vo1b], axis=1)
    vs_p = vo1[:N]
    vf_p = vo1[NP_PAD:NP_PAD + N]

    h2s, h2f, st2s, st2f = _enc2(vs_p, vf_p, ps["W2"], pf["W2"])
    sc2s, t2s = _affine(st2s[0], st2s[1], ps["g2"], ps["be2"])
    sc2f, t2f = _affine(st2f[0], st2f[1], pf["g2"], pf["be2"])
    aff2 = jnp.stack([sc2s, t2s, sc2f, t2f])
    h2cat = jnp.concatenate([h2s, h2f], axis=0)

    (vo2,) = _smooth_out(h2cat, vil, vig, eil, aff2)
    zs_p = vo2[:N]
    zf_p = vo2[NP_PAD:NP_PAD + N]

    z, zs, zf, xh = _fuse(zs_p, zf_p, params["att"], params["dec"])
    return (z, zs, zf, xh)
